# Initial kernel scaffold; baseline (speedup 1.0000x reference)
import jax
import jax.numpy as jnp
from jax.experimental import pallas as pl


def kernel(embed, emb_weight):
    raise NotImplementedError("write your pallas kernel here")


# speedup vs baseline: 2.5637x; 2.5637x over previous
import jax
import jax.numpy as jnp
from jax.experimental import pallas as pl

_BM = 512


def _readout_matmul_kernel(a_ref, w_ref, o_ref):
    a = a_ref[...].astype(jnp.bfloat16)
    w = w_ref[...].astype(jnp.bfloat16)
    o_ref[...] = jax.lax.dot_general(
        a, w,
        dimension_numbers=(((1,), (1,)), ((), ())),
        preferred_element_type=jnp.float32,
    )


def kernel(embed, emb_weight):
    m, d = embed.shape
    l, _ = emb_weight.shape
    grid = (m // _BM,)
    return pl.pallas_call(
        _readout_matmul_kernel,
        grid=grid,
        in_specs=[
            pl.BlockSpec((_BM, d), lambda i: (i, 0)),
            pl.BlockSpec((l, d), lambda i: (0, 0)),
        ],
        out_specs=pl.BlockSpec((_BM, l), lambda i: (i, 0)),
        out_shape=jax.ShapeDtypeStruct((m, l), jnp.float32),
    )(embed, emb_weight)
